# baseline (speedup 1.0000x reference)
import functools

import jax
import jax.numpy as jnp
from jax import lax
from jax.experimental import pallas as pl
from jax.experimental.pallas import tpu as pltpu
from jax.experimental.pallas import tpu_sc as plsc

N = 10000
E = 320000
H = 128
G = 64

NC = 2
NS = 16
NW = NC * NS
EPW = E // NW
EK = 125
EC = EPW // EK
ECH = EC // 2
RPS = 632
NACC = NS * RPS

_NEG = float(jnp.finfo(jnp.float32).min)


def _sc_scatter_body(hw_hbm, srcs_hbm, dsts_hbm, zeros_hbm, out_hbm,
                     src_v, dst_v, rows_v, accum, sem0, sem1, ssem0, ssem1):
  c = lax.axis_index("c")
  s = lax.axis_index("s")
  w = s * NC + c

  row0 = pl.multiple_of(s * RPS, 8)
  pltpu.sync_copy(zeros_hbm.at[pl.ds(row0, RPS)],
                  accum.at[pl.ds(row0, RPS)])
  plsc.subcore_barrier()

  sems = (sem0, sem1)
  ssems = (ssem0, ssem1)

  for half in range(2):
    pltpu.sync_copy(srcs_hbm.at[w, pl.ds(half * ECH, ECH)], src_v)
    pltpu.sync_copy(dsts_hbm.at[w, pl.ds(half * ECH, ECH)], dst_v)

    pltpu.async_copy(hw_hbm.at[src_v.at[0]], rows_v.at[0], sem0)

    def outer(jj, carry):
      for b in range(2):
        j = jj * 2 + b
        nb = (b + 1) % 2

        pltpu.make_async_copy(hw_hbm.at[src_v.at[j]], rows_v.at[b],
                              sems[b]).wait()
        pltpu.async_copy(rows_v.at[b], accum.at[dst_v.at[j]], ssems[b],
                         add=True)

        @pl.when(j + 1 < ECH)
        def _():
          @pl.when(j > 0)
          def _():
            pltpu.make_async_copy(rows_v.at[nb], accum.at[dst_v.at[j - 1]],
                                  ssems[nb]).wait()

          pltpu.async_copy(hw_hbm.at[src_v.at[j + 1]], rows_v.at[nb],
                           sems[nb])
      return carry

    lax.fori_loop(0, ECH // 2, outer, 0)
    pltpu.make_async_copy(rows_v.at[0], accum.at[dst_v.at[ECH - 2]],
                          ssems[0]).wait()
    pltpu.make_async_copy(rows_v.at[1], accum.at[dst_v.at[ECH - 1]],
                          ssems[1]).wait()

  plsc.subcore_barrier()
  pltpu.sync_copy(accum.at[pl.ds(row0, RPS)],
                  out_hbm.at[c, pl.ds(row0, RPS)])


def _make_sc_scatter():
  mesh = plsc.VectorSubcoreMesh(core_axis_name="c", subcore_axis_name="s")
  return pl.kernel(
      _sc_scatter_body,
      out_type=jax.ShapeDtypeStruct((NC, NACC, H), jnp.float32),
      mesh=mesh,
      scratch_types=[
          pltpu.VMEM((ECH, EK), jnp.int32),
          pltpu.VMEM((ECH, EK), jnp.int32),
          pltpu.VMEM((2, EK, H), jnp.float32),
          pltpu.VMEM_SHARED((NACC, H), jnp.float32),
          pltpu.SemaphoreType.DMA,
          pltpu.SemaphoreType.DMA,
          pltpu.SemaphoreType.DMA,
          pltpu.SemaphoreType.DMA,
      ],
      name="gcn_edge_scatter",
  )


R = 1000
NB = N // R


def _stage_a_body(x_ref, w1_ref, wres_ref, bres_ref, hw_ref, res_ref):
  xb = x_ref[...]
  hw_ref[...] = jnp.dot(xb, w1_ref[...], preferred_element_type=jnp.float32)
  res_ref[...] = jax.nn.relu(
      jnp.dot(xb, wres_ref[...], preferred_element_type=jnp.float32)
      + bres_ref[...])


def _stage_a(x, w1, wres, bres):
  return pl.pallas_call(
      _stage_a_body,
      grid=(NB,),
      in_specs=[
          pl.BlockSpec((R, H), lambda i: (i, 0)),
          pl.BlockSpec((H, H), lambda i: (0, 0)),
          pl.BlockSpec((H, H), lambda i: (0, 0)),
          pl.BlockSpec((1, H), lambda i: (0, 0)),
      ],
      out_specs=[
          pl.BlockSpec((R, H), lambda i: (i, 0)),
          pl.BlockSpec((R, H), lambda i: (i, 0)),
      ],
      out_shape=[
          jax.ShapeDtypeStruct((N, H), jnp.float32),
          jax.ShapeDtypeStruct((N, H), jnp.float32),
      ],
  )(x, w1, wres, bres)


def _stage_b_body(agg_ref, b_ref, res_ref, w2_ref, wres_ref, bres_ref,
                  hw2_ref, res2_ref):
  h = (jax.nn.relu(agg_ref[0] + agg_ref[1] + b_ref[...]) + res_ref[...])
  hw2_ref[...] = jnp.dot(h, w2_ref[...], preferred_element_type=jnp.float32)
  res2_ref[...] = jax.nn.relu(
      jnp.dot(h, wres_ref[...], preferred_element_type=jnp.float32)
      + bres_ref[...])


def _stage_b(agg, b, res, w2, wres, bres):
  return pl.pallas_call(
      _stage_b_body,
      grid=(NB,),
      in_specs=[
          pl.BlockSpec((NC, R, H), lambda i: (0, i, 0)),
          pl.BlockSpec((1, H), lambda i: (0, 0)),
          pl.BlockSpec((R, H), lambda i: (i, 0)),
          pl.BlockSpec((H, H), lambda i: (0, 0)),
          pl.BlockSpec((H, H), lambda i: (0, 0)),
          pl.BlockSpec((1, H), lambda i: (0, 0)),
      ],
      out_specs=[
          pl.BlockSpec((R, H), lambda i: (i, 0)),
          pl.BlockSpec((R, H), lambda i: (i, 0)),
      ],
      out_shape=[
          jax.ShapeDtypeStruct((N, H), jnp.float32),
          jax.ShapeDtypeStruct((N, H), jnp.float32),
      ],
  )(agg, b, res, w2, wres, bres)


def _stage_c_body(agg_ref, b_ref, res_ref, ids_row_ref, ids_col_ref,
                  watt_ref, batt_ref, sum_ref, max_ref):
  i = pl.program_id(0)
  h = (jax.nn.relu(agg_ref[0] + agg_ref[1] + b_ref[...]) + res_ref[...])
  att = jax.nn.sigmoid(
      jnp.dot(h, watt_ref[...], preferred_element_type=jnp.float32)
      + batt_ref[...])
  weighted = h * att

  ids_row = ids_row_ref[0]
  onehot_t = (lax.broadcasted_iota(jnp.int32, (G, R), 0)
              == ids_row).astype(jnp.float32)
  part_sum = jnp.dot(onehot_t, weighted, preferred_element_type=jnp.float32)

  ids_col = ids_col_ref[0]
  vals = h
  k = 1
  while k < R:
    vals_sh = jnp.concatenate(
        [jnp.full((k, H), _NEG, jnp.float32), vals[:R - k]], axis=0)
    ids_sh = jnp.concatenate(
        [jnp.full((k, 1), -1, jnp.int32), ids_col[:R - k]], axis=0)
    vals = jnp.where(ids_sh == ids_col, jnp.maximum(vals, vals_sh), vals)
    k *= 2

  nxt = jnp.concatenate(
      [ids_row[:, 1:], jnp.full((1, 1), -1, jnp.int32)], axis=1)
  m_sel = onehot_t * (ids_row != nxt).astype(jnp.float32)
  pres = jnp.dot(m_sel, jnp.full((R, 1), 1.0, jnp.float32),
                 preferred_element_type=jnp.float32)
  part_max = (jnp.dot(m_sel, vals, preferred_element_type=jnp.float32)
              + (pres - 1.0) * 3.0e38)

  @pl.when(i == 0)
  def _():
    sum_ref[...] = jnp.zeros_like(sum_ref)
    max_ref[...] = jnp.full_like(max_ref, _NEG)

  sum_ref[...] += part_sum
  max_ref[...] = jnp.maximum(max_ref[...], part_max)


def _stage_c(agg, b, res, ids_row, ids_col, watt, batt):
  return pl.pallas_call(
      _stage_c_body,
      grid=(NB,),
      in_specs=[
          pl.BlockSpec((NC, R, H), lambda i: (0, i, 0)),
          pl.BlockSpec((1, H), lambda i: (0, 0)),
          pl.BlockSpec((R, H), lambda i: (i, 0)),
          pl.BlockSpec((1, 1, R), lambda i: (i, 0, 0)),
          pl.BlockSpec((1, R, 1), lambda i: (i, 0, 0)),
          pl.BlockSpec((H, 1), lambda i: (0, 0)),
          pl.BlockSpec((1, 1), lambda i: (0, 0)),
      ],
      out_specs=[
          pl.BlockSpec((G, H), lambda i: (0, 0)),
          pl.BlockSpec((G, H), lambda i: (0, 0)),
      ],
      out_shape=[
          jax.ShapeDtypeStruct((G, H), jnp.float32),
          jax.ShapeDtypeStruct((G, H), jnp.float32),
      ],
  )(agg, b, res, ids_row, ids_col, watt, batt)


def _stage_d_body(s_ref, m_ref, wp1a_ref, wp1b_ref, bp1_ref, wp2_ref, bp2_ref,
                  out_ref):
  hid = jax.nn.relu(
      jnp.dot(s_ref[...], wp1a_ref[...], preferred_element_type=jnp.float32)
      + jnp.dot(m_ref[...], wp1b_ref[...], preferred_element_type=jnp.float32)
      + bp1_ref[...])
  out_ref[...] = (jnp.dot(hid, wp2_ref[...],
                          preferred_element_type=jnp.float32) + bp2_ref[...])


def _stage_d(seg_sum, seg_max, wp1a, wp1b, bp1, wp2, bp2):
  return pl.pallas_call(
      _stage_d_body,
      out_shape=jax.ShapeDtypeStruct((G, wp2.shape[1]), jnp.float32),
  )(seg_sum, seg_max, wp1a, wp1b, bp1, wp2, bp2)


@jax.jit
def kernel(x, edge_index, graph_ids, W1, b1, Wres1, bres1, W2, b2, Wres2,
           bres2, Watt, batt, Wp1, bp1, Wp2, bp2):
  srcs = edge_index[0].reshape(NW, EC, EK)
  dsts = edge_index[1].reshape(NW, EC, EK)
  zeros = jnp.zeros((NACC, H), jnp.float32)
  ids_row = graph_ids.reshape(NB, 1, R)
  ids_col = graph_ids.reshape(NB, R, 1)

  b1r = b1.reshape(1, H)
  bres1r = bres1.reshape(1, H)
  b2r = b2.reshape(1, H)
  bres2r = bres2.reshape(1, H)
  battr = batt.reshape(1, 1)
  bp1r = bp1.reshape(1, H)
  bp2r = bp2.reshape(1, Wp2.shape[1])
  wp1a = Wp1[:H]
  wp1b = Wp1[H:]

  sc_scatter = _make_sc_scatter()

  hw1, res1 = _stage_a(x, W1, Wres1, bres1r)
  agg1 = sc_scatter(hw1, srcs, dsts, zeros)
  hw2, res2 = _stage_b(agg1, b1r, res1, W2, Wres2, bres2r)
  agg2 = sc_scatter(hw2, srcs, dsts, zeros)
  seg_sum, seg_max = _stage_c(agg2, b2r, res2, ids_row, ids_col, Watt, battr)
  return _stage_d(seg_sum, seg_max, wp1a, wp1b, bp1r, Wp2, bp2r)


# speedup vs baseline: 1.1409x; 1.1409x over previous
import functools

import jax
import jax.numpy as jnp
from jax import lax
from jax.experimental import pallas as pl
from jax.experimental.pallas import tpu as pltpu
from jax.experimental.pallas import tpu_sc as plsc

N = 10000
E = 320000
H = 128
G = 64

NC = 2
NS = 16
NW = NC * NS
EPW = E // NW
EK = 125
EC = EPW // EK
ECH = EC // 2
RPS = 632
NACC = NS * RPS

_NEG = float(jnp.finfo(jnp.float32).min)


def _sc_scatter_body(hw_hbm, srcs_hbm, dsts_hbm, zeros_hbm, out_hbm,
                     src_v, dst_v, rows_v, accum, sem0, sem1):
  c = lax.axis_index("c")
  s = lax.axis_index("s")
  w = s * NC + c

  row0 = pl.multiple_of(s * RPS, 8)
  pltpu.sync_copy(zeros_hbm.at[pl.ds(row0, RPS)],
                  accum.at[pl.ds(row0, RPS)])
  plsc.subcore_barrier()

  sems = (sem0, sem1)

  for half in range(2):
    pltpu.sync_copy(srcs_hbm.at[w, pl.ds(half * ECH, ECH)], src_v)
    pltpu.sync_copy(dsts_hbm.at[w, pl.ds(half * ECH, ECH)], dst_v)

    pltpu.async_copy(hw_hbm.at[src_v.at[0]], rows_v.at[0], sem0)

    def outer(jj, carry):
      for b in range(2):
        j = jj * 2 + b

        @pl.when(j + 1 < ECH)
        def _():
          pltpu.async_copy(hw_hbm.at[src_v.at[j + 1]],
                           rows_v.at[(b + 1) % 2], sems[(b + 1) % 2])

        pltpu.make_async_copy(hw_hbm.at[src_v.at[j]], rows_v.at[b],
                              sems[b]).wait()
        pltpu.sync_copy(rows_v.at[b], accum.at[dst_v.at[j]], add=True)
      return carry

    lax.fori_loop(0, ECH // 2, outer, 0)

  plsc.subcore_barrier()
  pltpu.sync_copy(accum.at[pl.ds(row0, RPS)],
                  out_hbm.at[c, pl.ds(row0, RPS)])


def _make_sc_scatter():
  mesh = plsc.VectorSubcoreMesh(core_axis_name="c", subcore_axis_name="s")
  return pl.kernel(
      _sc_scatter_body,
      out_type=jax.ShapeDtypeStruct((NC, NACC, H), jnp.float32),
      mesh=mesh,
      scratch_types=[
          pltpu.VMEM((ECH, EK), jnp.int32),
          pltpu.VMEM((ECH, EK), jnp.int32),
          pltpu.VMEM((2, EK, H), jnp.float32),
          pltpu.VMEM_SHARED((NACC, H), jnp.float32),
          pltpu.SemaphoreType.DMA,
          pltpu.SemaphoreType.DMA,
      ],
      name="gcn_edge_scatter",
  )


R = 1000
NB = N // R


def _stage_a_body(x_ref, w1_ref, wres_ref, bres_ref, hw_ref, res_ref):
  xb = x_ref[...]
  hw_ref[...] = jnp.dot(xb, w1_ref[...], preferred_element_type=jnp.float32)
  res_ref[...] = jax.nn.relu(
      jnp.dot(xb, wres_ref[...], preferred_element_type=jnp.float32)
      + bres_ref[...])


def _stage_a(x, w1, wres, bres):
  return pl.pallas_call(
      _stage_a_body,
      grid=(NB,),
      in_specs=[
          pl.BlockSpec((R, H), lambda i: (i, 0)),
          pl.BlockSpec((H, H), lambda i: (0, 0)),
          pl.BlockSpec((H, H), lambda i: (0, 0)),
          pl.BlockSpec((1, H), lambda i: (0, 0)),
      ],
      out_specs=[
          pl.BlockSpec((R, H), lambda i: (i, 0)),
          pl.BlockSpec((R, H), lambda i: (i, 0)),
      ],
      out_shape=[
          jax.ShapeDtypeStruct((N, H), jnp.float32),
          jax.ShapeDtypeStruct((N, H), jnp.float32),
      ],
  )(x, w1, wres, bres)


def _stage_b_body(agg_ref, b_ref, res_ref, w2_ref, wres_ref, bres_ref,
                  hw2_ref, res2_ref):
  h = (jax.nn.relu(agg_ref[0] + agg_ref[1] + b_ref[...]) + res_ref[...])
  hw2_ref[...] = jnp.dot(h, w2_ref[...], preferred_element_type=jnp.float32)
  res2_ref[...] = jax.nn.relu(
      jnp.dot(h, wres_ref[...], preferred_element_type=jnp.float32)
      + bres_ref[...])


def _stage_b(agg, b, res, w2, wres, bres):
  return pl.pallas_call(
      _stage_b_body,
      grid=(NB,),
      in_specs=[
          pl.BlockSpec((NC, R, H), lambda i: (0, i, 0)),
          pl.BlockSpec((1, H), lambda i: (0, 0)),
          pl.BlockSpec((R, H), lambda i: (i, 0)),
          pl.BlockSpec((H, H), lambda i: (0, 0)),
          pl.BlockSpec((H, H), lambda i: (0, 0)),
          pl.BlockSpec((1, H), lambda i: (0, 0)),
      ],
      out_specs=[
          pl.BlockSpec((R, H), lambda i: (i, 0)),
          pl.BlockSpec((R, H), lambda i: (i, 0)),
      ],
      out_shape=[
          jax.ShapeDtypeStruct((N, H), jnp.float32),
          jax.ShapeDtypeStruct((N, H), jnp.float32),
      ],
  )(agg, b, res, w2, wres, bres)


def _stage_c_body(agg_ref, b_ref, res_ref, ids_row_ref, ids_col_ref,
                  watt_ref, batt_ref, sum_ref, max_ref):
  i = pl.program_id(0)
  h = (jax.nn.relu(agg_ref[0] + agg_ref[1] + b_ref[...]) + res_ref[...])
  att = jax.nn.sigmoid(
      jnp.dot(h, watt_ref[...], preferred_element_type=jnp.float32)
      + batt_ref[...])
  weighted = h * att

  ids_row = ids_row_ref[0]
  onehot_t = (lax.broadcasted_iota(jnp.int32, (G, R), 0)
              == ids_row).astype(jnp.float32)
  part_sum = jnp.dot(onehot_t, weighted, preferred_element_type=jnp.float32)

  ids_col = ids_col_ref[0]
  vals = h
  k = 1
  while k < R:
    vals_sh = jnp.concatenate(
        [jnp.full((k, H), _NEG, jnp.float32), vals[:R - k]], axis=0)
    ids_sh = jnp.concatenate(
        [jnp.full((k, 1), -1, jnp.int32), ids_col[:R - k]], axis=0)
    vals = jnp.where(ids_sh == ids_col, jnp.maximum(vals, vals_sh), vals)
    k *= 2

  nxt = jnp.concatenate(
      [ids_row[:, 1:], jnp.full((1, 1), -1, jnp.int32)], axis=1)
  m_sel = onehot_t * (ids_row != nxt).astype(jnp.float32)
  pres = jnp.dot(m_sel, jnp.full((R, 1), 1.0, jnp.float32),
                 preferred_element_type=jnp.float32)
  part_max = (jnp.dot(m_sel, vals, preferred_element_type=jnp.float32)
              + (pres - 1.0) * 3.0e38)

  @pl.when(i == 0)
  def _():
    sum_ref[...] = jnp.zeros_like(sum_ref)
    max_ref[...] = jnp.full_like(max_ref, _NEG)

  sum_ref[...] += part_sum
  max_ref[...] = jnp.maximum(max_ref[...], part_max)


def _stage_c(agg, b, res, ids_row, ids_col, watt, batt):
  return pl.pallas_call(
      _stage_c_body,
      grid=(NB,),
      in_specs=[
          pl.BlockSpec((NC, R, H), lambda i: (0, i, 0)),
          pl.BlockSpec((1, H), lambda i: (0, 0)),
          pl.BlockSpec((R, H), lambda i: (i, 0)),
          pl.BlockSpec((1, 1, R), lambda i: (i, 0, 0)),
          pl.BlockSpec((1, R, 1), lambda i: (i, 0, 0)),
          pl.BlockSpec((H, 1), lambda i: (0, 0)),
          pl.BlockSpec((1, 1), lambda i: (0, 0)),
      ],
      out_specs=[
          pl.BlockSpec((G, H), lambda i: (0, 0)),
          pl.BlockSpec((G, H), lambda i: (0, 0)),
      ],
      out_shape=[
          jax.ShapeDtypeStruct((G, H), jnp.float32),
          jax.ShapeDtypeStruct((G, H), jnp.float32),
      ],
  )(agg, b, res, ids_row, ids_col, watt, batt)


def _stage_d_body(s_ref, m_ref, wp1a_ref, wp1b_ref, bp1_ref, wp2_ref, bp2_ref,
                  out_ref):
  hid = jax.nn.relu(
      jnp.dot(s_ref[...], wp1a_ref[...], preferred_element_type=jnp.float32)
      + jnp.dot(m_ref[...], wp1b_ref[...], preferred_element_type=jnp.float32)
      + bp1_ref[...])
  out_ref[...] = (jnp.dot(hid, wp2_ref[...],
                          preferred_element_type=jnp.float32) + bp2_ref[...])


def _stage_d(seg_sum, seg_max, wp1a, wp1b, bp1, wp2, bp2):
  return pl.pallas_call(
      _stage_d_body,
      out_shape=jax.ShapeDtypeStruct((G, wp2.shape[1]), jnp.float32),
  )(seg_sum, seg_max, wp1a, wp1b, bp1, wp2, bp2)


@jax.jit
def kernel(x, edge_index, graph_ids, W1, b1, Wres1, bres1, W2, b2, Wres2,
           bres2, Watt, batt, Wp1, bp1, Wp2, bp2):
  srcs = edge_index[0].reshape(NW, EC, EK)
  dsts = edge_index[1].reshape(NW, EC, EK)
  zeros = jnp.zeros((NACC, H), jnp.float32)
  ids_row = graph_ids.reshape(NB, 1, R)
  ids_col = graph_ids.reshape(NB, R, 1)

  b1r = b1.reshape(1, H)
  bres1r = bres1.reshape(1, H)
  b2r = b2.reshape(1, H)
  bres2r = bres2.reshape(1, H)
  battr = batt.reshape(1, 1)
  bp1r = bp1.reshape(1, H)
  bp2r = bp2.reshape(1, Wp2.shape[1])
  wp1a = Wp1[:H]
  wp1b = Wp1[H:]

  sc_scatter = _make_sc_scatter()

  hw1, res1 = _stage_a(x, W1, Wres1, bres1r)
  agg1 = sc_scatter(hw1, srcs, dsts, zeros)
  hw2, res2 = _stage_b(agg1, b1r, res1, W2, Wres2, bres2r)
  agg2 = sc_scatter(hw2, srcs, dsts, zeros)
  seg_sum, seg_max = _stage_c(agg2, b2r, res2, ids_row, ids_col, Watt, battr)
  return _stage_d(seg_sum, seg_max, wp1a, wp1b, bp1r, Wp2, bp2r)


# speedup vs baseline: 1.1443x; 1.0030x over previous
import functools

import jax
import jax.numpy as jnp
from jax import lax
from jax.experimental import pallas as pl
from jax.experimental.pallas import tpu as pltpu
from jax.experimental.pallas import tpu_sc as plsc

N = 10000
E = 320000
H = 128
G = 64

NC = 2
NS = 16
NW = NC * NS
EPW = E // NW
EK = 125
EC = EPW // EK
ECH = EC // 2
RPS = 632
NACC = NS * RPS

_NEG = float(jnp.finfo(jnp.float32).min)


def _sc_scatter_body(hw_hbm, srcs_hbm, dsts_hbm, zeros_hbm, out_hbm,
                     src_v, dst_v, rows_v, accum, sem0, sem1):
  c = lax.axis_index("c")
  s = lax.axis_index("s")
  w = s * NC + c

  row0 = pl.multiple_of(s * RPS, 8)
  pltpu.sync_copy(zeros_hbm.at[pl.ds(row0, RPS)],
                  accum.at[pl.ds(row0, RPS)])
  plsc.subcore_barrier()

  sems = (sem0, sem1)

  for half in range(2):
    pltpu.sync_copy(srcs_hbm.at[w, pl.ds(half * ECH, ECH)], src_v)
    pltpu.sync_copy(dsts_hbm.at[w, pl.ds(half * ECH, ECH)], dst_v)

    pltpu.async_copy(hw_hbm.at[src_v.at[0]], rows_v.at[0], sem0)

    def outer(jj, carry):
      for b in range(2):
        j = jj * 2 + b

        @pl.when(j + 1 < ECH)
        def _():
          pltpu.async_copy(hw_hbm.at[src_v.at[j + 1]],
                           rows_v.at[(b + 1) % 2], sems[(b + 1) % 2])

        pltpu.make_async_copy(hw_hbm.at[src_v.at[j]], rows_v.at[b],
                              sems[b]).wait()
        pltpu.sync_copy(rows_v.at[b], accum.at[dst_v.at[j]], add=True)
      return carry

    lax.fori_loop(0, ECH // 2, outer, 0)

  plsc.subcore_barrier()
  pltpu.sync_copy(accum.at[pl.ds(row0, RPS)],
                  out_hbm.at[c, pl.ds(row0, RPS)])


def _make_sc_scatter():
  mesh = plsc.VectorSubcoreMesh(core_axis_name="c", subcore_axis_name="s")
  return pl.kernel(
      _sc_scatter_body,
      out_type=jax.ShapeDtypeStruct((NC, NACC, H), jnp.float32),
      mesh=mesh,
      scratch_types=[
          pltpu.VMEM((ECH, EK), jnp.int32),
          pltpu.VMEM((ECH, EK), jnp.int32),
          pltpu.VMEM((2, EK, H), jnp.float32),
          pltpu.VMEM_SHARED((NACC, H), jnp.float32),
          pltpu.SemaphoreType.DMA,
          pltpu.SemaphoreType.DMA,
      ],
      name="gcn_edge_scatter",
  )


R = 1000
NB = N // R


def _stage_a_body(x_ref, w1_ref, wres_ref, bres_ref, hw_ref, res_ref):
  xb = x_ref[...]
  hw_ref[...] = jnp.dot(xb, w1_ref[...], preferred_element_type=jnp.float32)
  res_ref[...] = jax.nn.relu(
      jnp.dot(xb, wres_ref[...], preferred_element_type=jnp.float32)
      + bres_ref[...])


def _stage_a(x, w1, wres, bres):
  return pl.pallas_call(
      _stage_a_body,
      grid=(NB,),
      in_specs=[
          pl.BlockSpec((R, H), lambda i: (i, 0)),
          pl.BlockSpec((H, H), lambda i: (0, 0)),
          pl.BlockSpec((H, H), lambda i: (0, 0)),
          pl.BlockSpec((1, H), lambda i: (0, 0)),
      ],
      out_specs=[
          pl.BlockSpec((R, H), lambda i: (i, 0)),
          pl.BlockSpec((R, H), lambda i: (i, 0)),
      ],
      out_shape=[
          jax.ShapeDtypeStruct((N, H), jnp.float32),
          jax.ShapeDtypeStruct((N, H), jnp.float32),
      ],
  )(x, w1, wres, bres)


def _stage_b_body(agg_ref, b_ref, res_ref, w2_ref, wres_ref, bres_ref,
                  hw2_ref, res2_ref):
  h = (jax.nn.relu(agg_ref[0] + agg_ref[1] + b_ref[...]) + res_ref[...])
  hw2_ref[...] = jnp.dot(h, w2_ref[...], preferred_element_type=jnp.float32)
  res2_ref[...] = jax.nn.relu(
      jnp.dot(h, wres_ref[...], preferred_element_type=jnp.float32)
      + bres_ref[...])


def _stage_b(agg, b, res, w2, wres, bres):
  return pl.pallas_call(
      _stage_b_body,
      grid=(NB,),
      in_specs=[
          pl.BlockSpec((NC, R, H), lambda i: (0, i, 0)),
          pl.BlockSpec((1, H), lambda i: (0, 0)),
          pl.BlockSpec((R, H), lambda i: (i, 0)),
          pl.BlockSpec((H, H), lambda i: (0, 0)),
          pl.BlockSpec((H, H), lambda i: (0, 0)),
          pl.BlockSpec((1, H), lambda i: (0, 0)),
      ],
      out_specs=[
          pl.BlockSpec((R, H), lambda i: (i, 0)),
          pl.BlockSpec((R, H), lambda i: (i, 0)),
      ],
      out_shape=[
          jax.ShapeDtypeStruct((N, H), jnp.float32),
          jax.ShapeDtypeStruct((N, H), jnp.float32),
      ],
  )(agg, b, res, w2, wres, bres)


def _stage_c_body(agg_ref, b_ref, res_ref, ids_row_ref, ids_col_ref,
                  watt_ref, batt_ref, wp1a_ref, wp1b_ref, bp1_ref, wp2_ref,
                  bp2_ref, out_ref, sum_ref, max_ref):
  i = pl.program_id(0)
  h = (jax.nn.relu(agg_ref[0] + agg_ref[1] + b_ref[...]) + res_ref[...])
  att = jax.nn.sigmoid(
      jnp.dot(h, watt_ref[...], preferred_element_type=jnp.float32)
      + batt_ref[...])
  weighted = h * att

  ids_row = ids_row_ref[0]
  onehot_t = (lax.broadcasted_iota(jnp.int32, (G, R), 0)
              == ids_row).astype(jnp.float32)
  part_sum = jnp.dot(onehot_t, weighted, preferred_element_type=jnp.float32)

  ids_col = ids_col_ref[0]
  vals = h
  k = 1
  while k < R:
    vals_sh = jnp.concatenate(
        [jnp.full((k, H), _NEG, jnp.float32), vals[:R - k]], axis=0)
    ids_sh = jnp.concatenate(
        [jnp.full((k, 1), -1, jnp.int32), ids_col[:R - k]], axis=0)
    vals = jnp.where(ids_sh == ids_col, jnp.maximum(vals, vals_sh), vals)
    k *= 2

  nxt = jnp.concatenate(
      [ids_row[:, 1:], jnp.full((1, 1), -1, jnp.int32)], axis=1)
  m_sel = onehot_t * (ids_row != nxt).astype(jnp.float32)
  pres = jnp.dot(m_sel, jnp.full((R, 1), 1.0, jnp.float32),
                 preferred_element_type=jnp.float32)
  part_max = (jnp.dot(m_sel, vals, preferred_element_type=jnp.float32)
              + (pres - 1.0) * 3.0e38)

  @pl.when(i == 0)
  def _():
    sum_ref[...] = jnp.zeros_like(sum_ref)
    max_ref[...] = jnp.full_like(max_ref, _NEG)

  sum_ref[...] += part_sum
  max_ref[...] = jnp.maximum(max_ref[...], part_max)

  @pl.when(i == NB - 1)
  def _():
    hid = jax.nn.relu(
        jnp.dot(sum_ref[...], wp1a_ref[...],
                preferred_element_type=jnp.float32)
        + jnp.dot(max_ref[...], wp1b_ref[...],
                  preferred_element_type=jnp.float32)
        + bp1_ref[...])
    out_ref[...] = (jnp.dot(hid, wp2_ref[...],
                            preferred_element_type=jnp.float32)
                    + bp2_ref[...])


def _stage_c(agg, b, res, ids_row, ids_col, watt, batt, wp1a, wp1b, bp1,
             wp2, bp2):
  nt = wp2.shape[1]
  return pl.pallas_call(
      _stage_c_body,
      grid=(NB,),
      in_specs=[
          pl.BlockSpec((NC, R, H), lambda i: (0, i, 0)),
          pl.BlockSpec((1, H), lambda i: (0, 0)),
          pl.BlockSpec((R, H), lambda i: (i, 0)),
          pl.BlockSpec((1, 1, R), lambda i: (i, 0, 0)),
          pl.BlockSpec((1, R, 1), lambda i: (i, 0, 0)),
          pl.BlockSpec((H, 1), lambda i: (0, 0)),
          pl.BlockSpec((1, 1), lambda i: (0, 0)),
          pl.BlockSpec((H, H), lambda i: (0, 0)),
          pl.BlockSpec((H, H), lambda i: (0, 0)),
          pl.BlockSpec((1, H), lambda i: (0, 0)),
          pl.BlockSpec((H, nt), lambda i: (0, 0)),
          pl.BlockSpec((1, nt), lambda i: (0, 0)),
      ],
      out_specs=pl.BlockSpec((G, nt), lambda i: (0, 0)),
      out_shape=jax.ShapeDtypeStruct((G, nt), jnp.float32),
      scratch_shapes=[
          pltpu.VMEM((G, H), jnp.float32),
          pltpu.VMEM((G, H), jnp.float32),
      ],
  )(agg, b, res, ids_row, ids_col, watt, batt, wp1a, wp1b, bp1, wp2, bp2)


@jax.jit
def kernel(x, edge_index, graph_ids, W1, b1, Wres1, bres1, W2, b2, Wres2,
           bres2, Watt, batt, Wp1, bp1, Wp2, bp2):
  srcs = edge_index[0].reshape(NW, EC, EK)
  dsts = edge_index[1].reshape(NW, EC, EK)
  zeros = jnp.zeros((NACC, H), jnp.float32)
  ids_row = graph_ids.reshape(NB, 1, R)
  ids_col = graph_ids.reshape(NB, R, 1)

  b1r = b1.reshape(1, H)
  bres1r = bres1.reshape(1, H)
  b2r = b2.reshape(1, H)
  bres2r = bres2.reshape(1, H)
  battr = batt.reshape(1, 1)
  bp1r = bp1.reshape(1, H)
  bp2r = bp2.reshape(1, Wp2.shape[1])
  wp1a = Wp1[:H]
  wp1b = Wp1[H:]

  sc_scatter = _make_sc_scatter()

  hw1, res1 = _stage_a(x, W1, Wres1, bres1r)
  agg1 = sc_scatter(hw1, srcs, dsts, zeros)
  hw2, res2 = _stage_b(agg1, b1r, res1, W2, Wres2, bres2r)
  agg2 = sc_scatter(hw2, srcs, dsts, zeros)
  return _stage_c(agg2, b2r, res2, ids_row, ids_col, Watt, battr,
                  wp1a, wp1b, bp1r, Wp2, bp2r)
